# Initial kernel scaffold; baseline (speedup 1.0000x reference)
import jax
import jax.numpy as jnp
from jax.experimental import pallas as pl


def kernel(x, edge_index, x1, W1, b1, W2, b2, Wl, bl):
    raise NotImplementedError("write your pallas kernel here")


# speedup vs baseline: 24.2885x; 24.2885x over previous
import functools

import jax
import jax.numpy as jnp
from jax import lax
from jax.experimental import pallas as pl
from jax.experimental.pallas import tpu as pltpu
from jax.experimental.pallas import tpu_sc as plsc

N = 10000
E = 320000
D = 128
NC = 2
NS = 16
NW = NC * NS
EPW = E // NW
B = 80
C = EPW // B
RPT = N // NS
GB = 1000
G = N // GB

_mesh = plsc.VectorSubcoreMesh(core_axis_name="c", subcore_axis_name="s")


@functools.partial(
    pl.kernel,
    out_type=jax.ShapeDtypeStruct((NC, NS, RPT, 16), jnp.float32),
    mesh=_mesh,
    compiler_params=pltpu.CompilerParams(use_tc_tiling_on_sc=False),
    scratch_types=[
        pltpu.VMEM((C, B), jnp.int32),
        pltpu.VMEM((B, 16), jnp.float32),
        pltpu.VMEM_SHARED((N, 16), jnp.float32),
    ],
)
def _deg_kernel(row_hbm, zeros_hbm, ones_hbm, out_hbm, idx_v, ones_v, acc_sh):
    c = lax.axis_index("c")
    s = lax.axis_index("s")
    wid = c * NS + s
    pltpu.sync_copy(zeros_hbm, acc_sh.at[pl.ds(s * RPT, RPT)])
    pltpu.sync_copy(row_hbm.at[wid], idx_v)
    pltpu.sync_copy(ones_hbm, ones_v)
    plsc.subcore_barrier()

    def body(j, _):
        pltpu.sync_copy(ones_v, acc_sh.at[idx_v.at[j]], add=True)
        return _

    lax.fori_loop(0, C, body, None)
    plsc.subcore_barrier()
    pltpu.sync_copy(acc_sh.at[pl.ds(s * RPT, RPT)], out_hbm.at[c, s])


def _make_conv_kernel(Dr):
    @functools.partial(
        pl.kernel,
        out_type=jax.ShapeDtypeStruct((NC, NS, RPT, Dr), jnp.float32),
        mesh=_mesh,
        compiler_params=pltpu.CompilerParams(use_tc_tiling_on_sc=False),
        scratch_types=[
            pltpu.VMEM((C, B), jnp.int32),
            pltpu.VMEM((C, B), jnp.int32),
            pltpu.VMEM((B, Dr), jnp.float32),
            pltpu.VMEM_SHARED((N, Dr), jnp.float32),
            pltpu.SemaphoreType.DMA,
        ],
    )
    def _conv_kernel(table_hbm, row_hbm, col_hbm, zeros_hbm, out_hbm,
                     row_v, col_v, buf_v, acc_sh, sem):
        c = lax.axis_index("c")
        s = lax.axis_index("s")
        wid = c * NS + s

        pltpu.sync_copy(zeros_hbm, acc_sh.at[pl.ds(s * RPT, RPT)])

        pltpu.sync_copy(row_hbm.at[wid], row_v)
        pltpu.sync_copy(col_hbm.at[wid], col_v)
        plsc.subcore_barrier()

        def body(j, _):
            pltpu.async_copy(table_hbm.at[row_v.at[j]], buf_v, sem).wait()
            pltpu.sync_copy(buf_v, acc_sh.at[col_v.at[j]], add=True)
            return _

        lax.fori_loop(0, C, body, None)
        plsc.subcore_barrier()
        pltpu.sync_copy(acc_sh.at[pl.ds(s * RPT, RPT)], out_hbm.at[c, s])

    return _conv_kernel


_conv32 = _make_conv_kernel(32)
_conv16 = _make_conv_kernel(16)


def _tc1_body(x_ref, w1_ref, b1_ref, dp_ref, ht_ref, dis_ref):
    xb = x_ref[...]
    h = lax.dot_general(xb, w1_ref[...], (((1,), (1,)), ((), ())),
                        preferred_element_type=jnp.float32) + b1_ref[...]
    dp = dp_ref[0]
    deg = dp[:, 0:1] + dp[:, 1:2] + 1.0
    dis = lax.rsqrt(deg)
    ht_ref[0] = h * dis
    dis_ref[0] = jnp.concatenate([dis, dis], axis=1)


def _tc2_body(p0_ref, p1_ref, ht1_ref, dis_ref, w2_ref, b2_ref, ht2_ref):
    acc = p0_ref[0] + p1_ref[0] + ht1_ref[0]
    dis = dis_ref[0][:, 0:1]
    out1 = jnp.maximum(dis * acc, 0.0)
    h2 = lax.dot_general(out1, w2_ref[...], (((1,), (1,)), ((), ())),
                         preferred_element_type=jnp.float32) + b2_ref[...]
    ht2 = h2 * dis
    ht2_ref[0] = jnp.concatenate(
        [ht2, jnp.zeros((GB, 8), jnp.float32)], axis=1)


def _tc3_body(q0_ref, q1_ref, htq_ref, dis_ref, x1_ref, wl_ref, bl_ref, z_ref, emb_ref):
    acc = (q0_ref[0] + q1_ref[0] + htq_ref[0])[:, 0:8]
    dis = dis_ref[0][:, 0:1]
    out2 = dis * acc
    m = jnp.max(out2, axis=1, keepdims=True)
    lse = jnp.log(jnp.sum(jnp.exp(out2 - m), axis=1, keepdims=True)) + m
    emb = out2 - lse
    wl = wl_ref[...]
    z = (jnp.sum(emb * wl[:, 0:8], axis=1, keepdims=True)
         + x1_ref[0] * wl[:, 8:9] + bl_ref[...])
    z_ref[0] = jnp.maximum(z, 0.0)
    emb_ref[0] = emb


def _blk(*shape):
    idx = lambda i: (i,) + (0,) * (len(shape) - 1)
    return pl.BlockSpec(shape, idx)


def _rep(*shape):
    idx = lambda i: (0,) * len(shape)
    return pl.BlockSpec(shape, idx)


def kernel(x, edge_index, x1, W1, b1, W2, b2, Wl, bl):
    row3 = edge_index[0].reshape(NW, C, B)
    col3 = edge_index[1].reshape(NW, C, B)
    zeros16 = jnp.zeros((RPT, 16), jnp.float32)
    zeros32 = jnp.zeros((RPT, 32), jnp.float32)
    ones_hbm = jnp.zeros((B, 16), jnp.float32).at[:, 0].set(1.0)

    degp = _deg_kernel(row3, zeros16, ones_hbm)
    dp = jnp.transpose(degp.reshape(NC, N, 16)[:, :, 0], (1, 0)).reshape(G, GB, 2)

    ht1, dis = pl.pallas_call(
        _tc1_body,
        grid=(G,),
        in_specs=[_blk(GB, D), _rep(32, D), _rep(1, 32), _blk(1, GB, 2)],
        out_specs=[_blk(1, GB, 32), _blk(1, GB, 2)],
        out_shape=[jax.ShapeDtypeStruct((G, GB, 32), jnp.float32),
                   jax.ShapeDtypeStruct((G, GB, 2), jnp.float32)],
    )(x, W1, b1.reshape(1, 32), dp)

    p = _conv32(ht1.reshape(N, 32), row3, col3, zeros32).reshape(NC, N, 32)

    ht2 = pl.pallas_call(
        _tc2_body,
        grid=(G,),
        in_specs=[_blk(1, GB, 32), _blk(1, GB, 32), _blk(1, GB, 32),
                  _blk(1, GB, 2), _rep(8, 32), _rep(1, 8)],
        out_specs=_blk(1, GB, 16),
        out_shape=jax.ShapeDtypeStruct((G, GB, 16), jnp.float32),
    )(p[0].reshape(G, GB, 32), p[1].reshape(G, GB, 32), ht1, dis,
      W2, b2.reshape(1, 8))

    q = _conv16(ht2.reshape(N, 16), row3, col3, zeros16).reshape(NC, N, 16)

    z, emb = pl.pallas_call(
        _tc3_body,
        grid=(G,),
        in_specs=[_blk(1, GB, 16), _blk(1, GB, 16), _blk(1, GB, 16),
                  _blk(1, GB, 2), _blk(1, GB, 1), _rep(1, 9), _rep(1, 1)],
        out_specs=[_blk(1, GB, 1), _blk(1, GB, 8)],
        out_shape=[jax.ShapeDtypeStruct((G, GB, 1), jnp.float32),
                   jax.ShapeDtypeStruct((G, GB, 8), jnp.float32)],
    )(q[0].reshape(G, GB, 16), q[1].reshape(G, GB, 16), ht2, dis,
      x1.reshape(G, GB, 1), Wl, bl.reshape(1, 1))

    return (z.reshape(N, 1), emb.reshape(N, 8))


# speedup vs baseline: 36.1001x; 1.4863x over previous
import functools

import jax
import jax.numpy as jnp
from jax import lax
from jax.experimental import pallas as pl
from jax.experimental.pallas import tpu as pltpu
from jax.experimental.pallas import tpu_sc as plsc

N = 10000
E = 320000
D = 128
NC = 2
NS = 16
NW = NC * NS
EPW = E // NW
B = 80
C = EPW // B
RPT = N // NS
GB = 1000
G = N // GB
KD = 25
KC = 5

_mesh = plsc.VectorSubcoreMesh(core_axis_name="c", subcore_axis_name="s")


@functools.partial(
    pl.kernel,
    out_type=jax.ShapeDtypeStruct((NC, NS, RPT, 16), jnp.float32),
    mesh=_mesh,
    compiler_params=pltpu.CompilerParams(use_tc_tiling_on_sc=False),
    scratch_types=[
        pltpu.VMEM((C, B), jnp.int32),
        pltpu.VMEM((B, 16), jnp.float32),
        pltpu.VMEM_SHARED((N, 16), jnp.float32),
        pltpu.SemaphoreType.DMA,
    ],
)
def _deg_kernel(row_hbm, zeros_hbm, ones_hbm, out_hbm, idx_v, ones_v, acc_sh,
                sem):
    c = lax.axis_index("c")
    s = lax.axis_index("s")
    wid = c * NS + s
    pltpu.sync_copy(zeros_hbm, acc_sh.at[pl.ds(s * RPT, RPT)])
    pltpu.sync_copy(row_hbm.at[wid], idx_v)
    pltpu.sync_copy(ones_hbm, ones_v)
    plsc.subcore_barrier()

    def body(i, _):
        descs = [
            pltpu.async_copy(ones_v, acc_sh.at[idx_v.at[i * KD + k]],
                             sem, add=True)
            for k in range(KD)
        ]
        for d in descs:
            d.wait()
        return _

    lax.fori_loop(0, C // KD, body, None)
    plsc.subcore_barrier()
    pltpu.sync_copy(acc_sh.at[pl.ds(s * RPT, RPT)], out_hbm.at[c, s])


def _make_conv_kernel(Dr):
    @functools.partial(
        pl.kernel,
        out_type=jax.ShapeDtypeStruct((NC, NS, RPT, Dr), jnp.float32),
        mesh=_mesh,
        compiler_params=pltpu.CompilerParams(use_tc_tiling_on_sc=False),
        scratch_types=[
            pltpu.VMEM((C, B), jnp.int32),
            pltpu.VMEM((C, B), jnp.int32),
            pltpu.VMEM((KC, B, Dr), jnp.float32),
            pltpu.VMEM_SHARED((N, Dr), jnp.float32),
            pltpu.SemaphoreType.DMA,
            pltpu.SemaphoreType.DMA,
        ],
    )
    def _conv_kernel(table_hbm, row_hbm, col_hbm, zeros_hbm, out_hbm,
                     row_v, col_v, buf_v, acc_sh, sem_g, sem_s):
        c = lax.axis_index("c")
        s = lax.axis_index("s")
        wid = c * NS + s

        pltpu.sync_copy(zeros_hbm, acc_sh.at[pl.ds(s * RPT, RPT)])

        pltpu.sync_copy(row_hbm.at[wid], row_v)
        pltpu.sync_copy(col_hbm.at[wid], col_v)
        plsc.subcore_barrier()

        def body(i, _):
            gd = [
                pltpu.async_copy(table_hbm.at[row_v.at[i * KC + k]],
                                 buf_v.at[k], sem_g)
                for k in range(KC)
            ]
            for d in gd:
                d.wait()
            sd = [
                pltpu.async_copy(buf_v.at[k], acc_sh.at[col_v.at[i * KC + k]],
                                 sem_s, add=True)
                for k in range(KC)
            ]
            for d in sd:
                d.wait()
            return _

        lax.fori_loop(0, C // KC, body, None)
        plsc.subcore_barrier()
        pltpu.sync_copy(acc_sh.at[pl.ds(s * RPT, RPT)], out_hbm.at[c, s])

    return _conv_kernel


_conv32 = _make_conv_kernel(32)
_conv16 = _make_conv_kernel(16)


def _tc1_body(x_ref, w1_ref, b1_ref, dp_ref, ht_ref, dis_ref):
    xb = x_ref[...]
    h = lax.dot_general(xb, w1_ref[...], (((1,), (1,)), ((), ())),
                        preferred_element_type=jnp.float32) + b1_ref[...]
    dp = dp_ref[0]
    deg = dp[:, 0:1] + dp[:, 1:2] + 1.0
    dis = lax.rsqrt(deg)
    ht_ref[0] = h * dis
    dis_ref[0] = jnp.concatenate([dis, dis], axis=1)


def _tc2_body(p0_ref, p1_ref, ht1_ref, dis_ref, w2_ref, b2_ref, ht2_ref):
    acc = p0_ref[0] + p1_ref[0] + ht1_ref[0]
    dis = dis_ref[0][:, 0:1]
    out1 = jnp.maximum(dis * acc, 0.0)
    h2 = lax.dot_general(out1, w2_ref[...], (((1,), (1,)), ((), ())),
                         preferred_element_type=jnp.float32) + b2_ref[...]
    ht2 = h2 * dis
    ht2_ref[0] = jnp.concatenate(
        [ht2, jnp.zeros((GB, 8), jnp.float32)], axis=1)


def _tc3_body(q0_ref, q1_ref, htq_ref, dis_ref, x1_ref, wl_ref, bl_ref, z_ref, emb_ref):
    acc = (q0_ref[0] + q1_ref[0] + htq_ref[0])[:, 0:8]
    dis = dis_ref[0][:, 0:1]
    out2 = dis * acc
    m = jnp.max(out2, axis=1, keepdims=True)
    lse = jnp.log(jnp.sum(jnp.exp(out2 - m), axis=1, keepdims=True)) + m
    emb = out2 - lse
    wl = wl_ref[...]
    z = (jnp.sum(emb * wl[:, 0:8], axis=1, keepdims=True)
         + x1_ref[0] * wl[:, 8:9] + bl_ref[...])
    z_ref[0] = jnp.maximum(z, 0.0)
    emb_ref[0] = emb


def _blk(*shape):
    idx = lambda i: (i,) + (0,) * (len(shape) - 1)
    return pl.BlockSpec(shape, idx)


def _rep(*shape):
    idx = lambda i: (0,) * len(shape)
    return pl.BlockSpec(shape, idx)


def kernel(x, edge_index, x1, W1, b1, W2, b2, Wl, bl):
    row3 = edge_index[0].reshape(NW, C, B)
    col3 = edge_index[1].reshape(NW, C, B)
    zeros16 = jnp.zeros((RPT, 16), jnp.float32)
    zeros32 = jnp.zeros((RPT, 32), jnp.float32)
    ones_hbm = jnp.zeros((B, 16), jnp.float32).at[:, 0].set(1.0)

    degp = _deg_kernel(row3, zeros16, ones_hbm)
    dp = jnp.transpose(degp.reshape(NC, N, 16)[:, :, 0], (1, 0)).reshape(G, GB, 2)

    ht1, dis = pl.pallas_call(
        _tc1_body,
        grid=(G,),
        in_specs=[_blk(GB, D), _rep(32, D), _rep(1, 32), _blk(1, GB, 2)],
        out_specs=[_blk(1, GB, 32), _blk(1, GB, 2)],
        out_shape=[jax.ShapeDtypeStruct((G, GB, 32), jnp.float32),
                   jax.ShapeDtypeStruct((G, GB, 2), jnp.float32)],
    )(x, W1, b1.reshape(1, 32), dp)

    p = _conv32(ht1.reshape(N, 32), row3, col3, zeros32).reshape(NC, N, 32)

    ht2 = pl.pallas_call(
        _tc2_body,
        grid=(G,),
        in_specs=[_blk(1, GB, 32), _blk(1, GB, 32), _blk(1, GB, 32),
                  _blk(1, GB, 2), _rep(8, 32), _rep(1, 8)],
        out_specs=_blk(1, GB, 16),
        out_shape=jax.ShapeDtypeStruct((G, GB, 16), jnp.float32),
    )(p[0].reshape(G, GB, 32), p[1].reshape(G, GB, 32), ht1, dis,
      W2, b2.reshape(1, 8))

    q = _conv16(ht2.reshape(N, 16), row3, col3, zeros16).reshape(NC, N, 16)

    z, emb = pl.pallas_call(
        _tc3_body,
        grid=(G,),
        in_specs=[_blk(1, GB, 16), _blk(1, GB, 16), _blk(1, GB, 16),
                  _blk(1, GB, 2), _blk(1, GB, 1), _rep(1, 9), _rep(1, 1)],
        out_specs=[_blk(1, GB, 1), _blk(1, GB, 8)],
        out_shape=[jax.ShapeDtypeStruct((G, GB, 1), jnp.float32),
                   jax.ShapeDtypeStruct((G, GB, 8), jnp.float32)],
    )(q[0].reshape(G, GB, 16), q[1].reshape(G, GB, 16), ht2, dis,
      x1.reshape(G, GB, 1), Wl, bl.reshape(1, 1))

    return (z.reshape(N, 1), emb.reshape(N, 8))


# speedup vs baseline: 39.9682x; 1.1072x over previous
import functools

import jax
import jax.numpy as jnp
from jax import lax
from jax.experimental import pallas as pl
from jax.experimental.pallas import tpu as pltpu
from jax.experimental.pallas import tpu_sc as plsc

N = 10000
E = 320000
D = 128
NC = 2
NS = 16
NW = NC * NS
EPW = E // NW
B = 80
C = EPW // B
RPT = N // NS
KD = 25
KC = 5

_mesh = plsc.VectorSubcoreMesh(core_axis_name="c", subcore_axis_name="s")
_sc_params = pltpu.CompilerParams(use_tc_tiling_on_sc=False)


@functools.partial(
    pl.kernel,
    out_type=jax.ShapeDtypeStruct((NC, NS, RPT, 32), jnp.float32),
    mesh=_mesh,
    compiler_params=_sc_params,
    scratch_types=[
        pltpu.VMEM((C, B), jnp.int32),
        pltpu.VMEM((B, 32), jnp.float32),
        pltpu.VMEM_SHARED((N, 32), jnp.float32),
        pltpu.SemaphoreType.DMA,
    ],
)
def _deg_kernel(row_hbm, zeros_hbm, ones_hbm, out_hbm, idx_v, ones_v, acc_sh,
                sem):
    c = lax.axis_index("c")
    s = lax.axis_index("s")
    wid = c * NS + s
    pltpu.sync_copy(zeros_hbm, acc_sh.at[pl.ds(s * RPT, RPT)])
    pltpu.sync_copy(row_hbm.at[wid], idx_v)
    pltpu.sync_copy(ones_hbm, ones_v)
    plsc.subcore_barrier()

    def body(i, _):
        descs = [
            pltpu.async_copy(ones_v, acc_sh.at[idx_v.at[i * KD + k]],
                             sem, add=True)
            for k in range(KD)
        ]
        for d in descs:
            d.wait()
        return _

    lax.fori_loop(0, C // KD, body, None)
    plsc.subcore_barrier()
    pltpu.sync_copy(acc_sh.at[pl.ds(s * RPT, RPT)], out_hbm.at[c, s])


def _make_conv_kernel(Dr):
    @functools.partial(
        pl.kernel,
        out_type=jax.ShapeDtypeStruct((NC, NS, RPT, Dr), jnp.float32),
        mesh=_mesh,
        compiler_params=_sc_params,
        scratch_types=[
            pltpu.VMEM((C, B), jnp.int32),
            pltpu.VMEM((C, B), jnp.int32),
            pltpu.VMEM((KC, B, Dr), jnp.float32),
            pltpu.VMEM_SHARED((N, Dr), jnp.float32),
            pltpu.SemaphoreType.DMA,
            pltpu.SemaphoreType.DMA,
        ],
    )
    def _conv_kernel(table_hbm, row_hbm, col_hbm, zeros_hbm, out_hbm,
                     row_v, col_v, buf_v, acc_sh, sem_g, sem_s):
        c = lax.axis_index("c")
        s = lax.axis_index("s")
        wid = c * NS + s

        pltpu.sync_copy(zeros_hbm, acc_sh.at[pl.ds(s * RPT, RPT)])
        pltpu.sync_copy(row_hbm.at[wid], row_v)
        pltpu.sync_copy(col_hbm.at[wid], col_v)
        plsc.subcore_barrier()

        def body(i, _):
            gd = [
                pltpu.async_copy(table_hbm.at[row_v.at[i * KC + k]],
                                 buf_v.at[k], sem_g)
                for k in range(KC)
            ]
            for d in gd:
                d.wait()
            sd = [
                pltpu.async_copy(buf_v.at[k], acc_sh.at[col_v.at[i * KC + k]],
                                 sem_s, add=True)
                for k in range(KC)
            ]
            for d in sd:
                d.wait()
            return _

        lax.fori_loop(0, C // KC, body, None)
        plsc.subcore_barrier()
        pltpu.sync_copy(acc_sh.at[pl.ds(s * RPT, RPT)], out_hbm.at[c, s])

    return _conv_kernel


_conv32 = _make_conv_kernel(32)
_conv8 = _make_conv_kernel(8)


def _iota2(shape, dim):
    return lax.broadcasted_iota(jnp.int32, shape, dim)


def _dis32(d0_ref, d1_ref):
    i = _iota2((128, 128), 0)
    j = _iota2((128, 128), 1)
    r32 = ((i % 32 == 0) & (j // 32 == i // 32)).astype(jnp.float32)
    dsum = jnp.dot(d0_ref[...] + d1_ref[...], r32,
                   preferred_element_type=jnp.float32)
    return lax.rsqrt(dsum + 1.0)


def _tc1_body(xp_ref, bd1_ref, b1p_ref, d0_ref, d1_ref, ht_ref):
    h = jnp.dot(xp_ref[...], bd1_ref[...],
                preferred_element_type=jnp.float32) + b1p_ref[...]
    ht_ref[...] = h * _dis32(d0_ref, d1_ref)


def _tc2_body(p0_ref, p1_ref, ht1_ref, d0_ref, d1_ref, bd2_ref, b2q_ref,
              ht2_ref, dis8_ref):
    dis32 = _dis32(d0_ref, d1_ref)
    s = p0_ref[...] + p1_ref[...] + ht1_ref[...]
    out1 = jnp.maximum(dis32 * s, 0.0)
    h2 = jnp.dot(out1, bd2_ref[...],
                 preferred_element_type=jnp.float32) + b2q_ref[...]
    i = _iota2((128, 32), 0)
    j = _iota2((128, 32), 1)
    s8 = (i == 32 * (j // 8)).astype(jnp.float32)
    dis8 = jnp.dot(dis32, s8, preferred_element_type=jnp.float32)
    ht2_ref[...] = h2 * dis8
    dis8_ref[...] = dis8


def _tc3_body(q0_ref, q1_ref, ht2_ref, dis8_ref, x1p_ref, wl_ref, bl_ref,
              z_ref, emb_ref):
    s = q0_ref[...] + q1_ref[...] + ht2_ref[...]
    out2 = dis8_ref[...] * s
    m = jnp.max(out2, axis=1, keepdims=True)
    e = jnp.exp(out2 - m)
    i = _iota2((128, 128), 0)
    j = _iota2((128, 128), 1)
    g8 = ((i // 8) == (j // 8)).astype(jnp.float32)
    ssum = jnp.dot(e, g8, preferred_element_type=jnp.float32)
    emb = (out2 - m) - jnp.log(ssum)
    wl = wl_ref[...]
    it = _iota2((8, 128), 0)
    jt = _iota2((8, 128), 1)
    tile8 = (jt % 8 == it).astype(jnp.float32)
    wlp = jnp.dot(wl[:, 0:8], tile8, preferred_element_type=jnp.float32)
    ig = _iota2((128, 16), 0)
    jg = _iota2((128, 16), 1)
    gsel = ((ig // 8) == jg).astype(jnp.float32)
    zq = jnp.dot(emb * wlp, gsel, preferred_element_type=jnp.float32)
    z = zq + x1p_ref[...] * wl[:, 8:9] + bl_ref[...]
    z_ref[...] = jnp.maximum(z, 0.0)
    emb_ref[...] = emb


def kernel(x, edge_index, x1, W1, b1, W2, b2, Wl, bl):
    f32 = jnp.float32
    row3 = edge_index[0].reshape(NW, C, B)
    col3 = edge_index[1].reshape(NW, C, B)
    zeros32 = jnp.zeros((RPT, 32), f32)
    zeros8 = jnp.zeros((RPT, 8), f32)
    ones_hbm = jnp.zeros((B, 32), f32).at[:, 0].set(1.0)

    degp = _deg_kernel(row3, zeros32, ones_hbm)
    d0p = degp[0].reshape(N // 4, 128)
    d1p = degp[1].reshape(N // 4, 128)

    bd1 = jax.scipy.linalg.block_diag(*([W1.T] * 4))
    bd2 = jax.scipy.linalg.block_diag(*([W2.T] * 4))
    b1p = jnp.tile(b1, 4).reshape(1, 128)
    b2q = jnp.tile(b2, 4).reshape(1, 32)

    ht1p = pl.pallas_call(
        _tc1_body,
        out_shape=jax.ShapeDtypeStruct((N // 4, 128), f32),
    )(x.reshape(N // 4, 512), bd1, b1p, d0p, d1p)

    p = _conv32(ht1p.reshape(N, 32), row3, col3, zeros32)

    ht2q, dis8q = pl.pallas_call(
        _tc2_body,
        out_shape=[jax.ShapeDtypeStruct((N // 4, 32), f32),
                   jax.ShapeDtypeStruct((N // 4, 32), f32)],
    )(p[0].reshape(N // 4, 128), p[1].reshape(N // 4, 128), ht1p,
      d0p, d1p, bd2, b2q)

    ht2lin = ht2q.reshape(N, 8)
    q = _conv8(ht2lin, row3, col3, zeros8)

    z16, embp = pl.pallas_call(
        _tc3_body,
        out_shape=[jax.ShapeDtypeStruct((N // 16, 16), f32),
                   jax.ShapeDtypeStruct((N // 16, 128), f32)],
    )(q[0].reshape(N // 16, 128), q[1].reshape(N // 16, 128),
      ht2lin.reshape(N // 16, 128), dis8q.reshape(N // 16, 128),
      x1.reshape(N // 16, 16), Wl, bl.reshape(1, 1))

    return (z16.reshape(N, 1), embp.reshape(N, 8))


# speedup vs baseline: 42.7167x; 1.0688x over previous
import functools

import jax
import jax.numpy as jnp
from jax import lax
from jax.experimental import pallas as pl
from jax.experimental.pallas import tpu as pltpu
from jax.experimental.pallas import tpu_sc as plsc

N = 10000
E = 320000
D = 128
NC = 2
NS = 16
NW = NC * NS
EPW = E // NW
B = 80
C = EPW // B
RPT = N // NS
KD = 25
KC = 5

_mesh = plsc.VectorSubcoreMesh(core_axis_name="c", subcore_axis_name="s")
_sc_params = pltpu.CompilerParams(use_tc_tiling_on_sc=False)


@functools.partial(
    pl.kernel,
    out_type=jax.ShapeDtypeStruct((NC, NS, RPT, 32), jnp.float32),
    mesh=_mesh,
    compiler_params=_sc_params,
    scratch_types=[
        pltpu.VMEM((C, B), jnp.int32),
        pltpu.VMEM((B, 32), jnp.float32),
        pltpu.VMEM_SHARED((N, 32), jnp.float32),
        pltpu.SemaphoreType.DMA,
    ],
)
def _deg_kernel(row_hbm, zeros_hbm, ones_hbm, out_hbm, idx_v, ones_v, acc_sh,
                sem):
    c = lax.axis_index("c")
    s = lax.axis_index("s")
    wid = c * NS + s
    pltpu.sync_copy(zeros_hbm, acc_sh.at[pl.ds(s * RPT, RPT)])
    pltpu.sync_copy(row_hbm.at[wid], idx_v)
    pltpu.sync_copy(ones_hbm, ones_v)
    plsc.subcore_barrier()

    def body(i, _):
        descs = [
            pltpu.async_copy(ones_v, acc_sh.at[idx_v.at[i * KD + k]],
                             sem, add=True)
            for k in range(KD)
        ]
        for d in descs:
            d.wait()
        return _

    lax.fori_loop(0, C // KD, body, None)
    plsc.subcore_barrier()
    pltpu.sync_copy(acc_sh.at[pl.ds(s * RPT, RPT)], out_hbm.at[c, s])


def _make_conv_kernel(Dr):
    @functools.partial(
        pl.kernel,
        out_type=jax.ShapeDtypeStruct((NC, NS, RPT, Dr), jnp.float32),
        mesh=_mesh,
        compiler_params=_sc_params,
        scratch_types=[
            pltpu.VMEM((C, B), jnp.int32),
            pltpu.VMEM((C, B), jnp.int32),
            pltpu.VMEM((KC, B, Dr), jnp.float32),
            pltpu.VMEM_SHARED((N, Dr), jnp.float32),
            pltpu.SemaphoreType.DMA,
            pltpu.SemaphoreType.DMA,
        ],
    )
    def _conv_kernel(table_hbm, row_hbm, col_hbm, zeros_hbm, out_hbm,
                     row_v, col_v, buf_v, acc_sh, sem_g, sem_s):
        c = lax.axis_index("c")
        s = lax.axis_index("s")
        wid = c * NS + s

        pltpu.sync_copy(zeros_hbm, acc_sh.at[pl.ds(s * RPT, RPT)])
        pltpu.sync_copy(row_hbm.at[wid], row_v)
        pltpu.sync_copy(col_hbm.at[wid], col_v)
        plsc.subcore_barrier()

        def body(i, _):
            gd = [
                pltpu.async_copy(table_hbm.at[row_v.at[i * KC + k]],
                                 buf_v.at[k], sem_g)
                for k in range(KC)
            ]
            for d in gd:
                d.wait()
            sd = [
                pltpu.async_copy(buf_v.at[k], acc_sh.at[col_v.at[i * KC + k]],
                                 sem_s, add=True)
                for k in range(KC)
            ]
            for d in sd:
                d.wait()
            return _

        lax.fori_loop(0, C // KC, body, None)
        plsc.subcore_barrier()
        pltpu.sync_copy(acc_sh.at[pl.ds(s * RPT, RPT)], out_hbm.at[c, s])

    return _conv_kernel


_conv32 = _make_conv_kernel(32)
_conv8 = _make_conv_kernel(8)


def _iota2(shape, dim):
    return lax.broadcasted_iota(jnp.int32, shape, dim)


def _dis32(d0_ref, d1_ref):
    i = _iota2((128, 128), 0)
    j = _iota2((128, 128), 1)
    r32 = ((i % 32 == 0) & (j // 32 == i // 32)).astype(jnp.float32)
    dsum = jnp.dot(d0_ref[...] + d1_ref[...], r32,
                   preferred_element_type=jnp.float32)
    return lax.rsqrt(dsum + 1.0)


def _tc1_body(xp_ref, bd1_ref, b1p_ref, d0_ref, d1_ref, ht_ref):
    h = jnp.dot(xp_ref[...], bd1_ref[...],
                preferred_element_type=jnp.float32) + b1p_ref[...]
    ht_ref[...] = h * _dis32(d0_ref, d1_ref)


def _tc2_body(p0_ref, p1_ref, ht1_ref, d0_ref, d1_ref, bd2_ref, b2q_ref,
              ht2_ref, dis8_ref):
    dis32 = _dis32(d0_ref, d1_ref)
    s = p0_ref[...] + p1_ref[...] + ht1_ref[...]
    out1 = jnp.maximum(dis32 * s, 0.0)
    h2 = jnp.dot(out1, bd2_ref[...],
                 preferred_element_type=jnp.float32) + b2q_ref[...]
    i = _iota2((128, 32), 0)
    j = _iota2((128, 32), 1)
    s8 = (i == 32 * (j // 8)).astype(jnp.float32)
    dis8 = jnp.dot(dis32, s8, preferred_element_type=jnp.float32)
    ht2_ref[...] = h2 * dis8
    dis8_ref[...] = dis8


def _tc3_body(q0_ref, q1_ref, ht2_ref, dis8_ref, x1p_ref, wl_ref, bl_ref,
              z_ref, emb_ref):
    s = q0_ref[...] + q1_ref[...] + ht2_ref[...]
    out2 = dis8_ref[...] * s
    m = jnp.max(out2, axis=1, keepdims=True)
    e = jnp.exp(out2 - m)
    i = _iota2((128, 128), 0)
    j = _iota2((128, 128), 1)
    g8 = ((i // 8) == (j // 8)).astype(jnp.float32)
    ssum = jnp.dot(e, g8, preferred_element_type=jnp.float32)
    emb = (out2 - m) - jnp.log(ssum)
    wl = wl_ref[...]
    it = _iota2((8, 128), 0)
    jt = _iota2((8, 128), 1)
    tile8 = (jt % 8 == it).astype(jnp.float32)
    wlp = jnp.dot(wl[:, 0:8], tile8, preferred_element_type=jnp.float32)
    ig = _iota2((128, 16), 0)
    jg = _iota2((128, 16), 1)
    gsel = ((ig // 8) == jg).astype(jnp.float32)
    zq = jnp.dot(emb * wlp, gsel, preferred_element_type=jnp.float32)
    z = zq + x1p_ref[...] * wl[:, 8:9] + bl_ref[...]
    z_ref[...] = jnp.maximum(z, 0.0)
    emb_ref[...] = emb


def kernel(x, edge_index, x1, W1, b1, W2, b2, Wl, bl):
    f32 = jnp.float32
    row3 = edge_index[0].reshape(NW, C, B)
    col3 = edge_index[1].reshape(NW, C, B)
    zeros32 = jnp.zeros((RPT, 32), f32)
    zeros8 = jnp.zeros((RPT, 8), f32)
    ones_hbm = jnp.zeros((B, 32), f32).at[:, 0].set(1.0)

    degp = _deg_kernel(row3, zeros32, ones_hbm)
    degp3 = degp.reshape(NC, N // 4, 128)
    d0p, d1p = degp3[0], degp3[1]

    bd1 = jax.scipy.linalg.block_diag(*([W1.T] * 4))
    bd2 = jax.scipy.linalg.block_diag(*([W2.T] * 4))
    b1p = jnp.tile(b1, 4).reshape(1, 128)
    b2q = jnp.tile(b2, 4).reshape(1, 32)

    ht1p = pl.pallas_call(
        _tc1_body,
        out_shape=jax.ShapeDtypeStruct((N // 4, 128), f32),
    )(x.reshape(N // 4, 512), bd1, b1p, d0p, d1p)

    p3 = _conv32(ht1p.reshape(N, 32), row3, col3,
                 zeros32).reshape(NC, N // 4, 128)

    ht2q, dis8q = pl.pallas_call(
        _tc2_body,
        out_shape=[jax.ShapeDtypeStruct((N // 4, 32), f32),
                   jax.ShapeDtypeStruct((N // 4, 32), f32)],
    )(p3[0], p3[1], ht1p, d0p, d1p, bd2, b2q)

    ht2lin = ht2q.reshape(N, 8)
    q3 = _conv8(ht2lin, row3, col3, zeros8).reshape(NC, N // 16, 128)

    z16, embp = pl.pallas_call(
        _tc3_body,
        out_shape=[jax.ShapeDtypeStruct((N // 16, 16), f32),
                   jax.ShapeDtypeStruct((N // 16, 128), f32)],
    )(q3[0], q3[1],
      ht2q.reshape(N // 16, 128), dis8q.reshape(N // 16, 128),
      x1.reshape(N // 16, 16), Wl, bl.reshape(1, 1))

    return (z16.reshape(N, 1), embp.reshape(N, 8))


# speedup vs baseline: 50.2156x; 1.1756x over previous
import functools

import jax
import jax.numpy as jnp
from jax import lax
from jax.experimental import pallas as pl
from jax.experimental.pallas import tpu as pltpu
from jax.experimental.pallas import tpu_sc as plsc

N = 10000
E = 320000
D = 128
NC = 2
NS = 16
NW = NC * NS
EPW = E // NW
B = 80
C = EPW // B
RPT = N // NS
KD = 25
KC = 5

_mesh = plsc.VectorSubcoreMesh(core_axis_name="c", subcore_axis_name="s")
_sc_params = pltpu.CompilerParams(use_tc_tiling_on_sc=False)


@functools.partial(
    pl.kernel,
    out_type=jax.ShapeDtypeStruct((NC, NS, RPT, 32), jnp.float32),
    mesh=_mesh,
    compiler_params=_sc_params,
    scratch_types=[
        pltpu.VMEM((C, B), jnp.int32),
        pltpu.VMEM((B, 32), jnp.float32),
        pltpu.VMEM_SHARED((N, 32), jnp.float32),
        pltpu.SemaphoreType.DMA,
    ],
)
def _deg_kernel(row_hbm, zeros_hbm, ones_hbm, out_hbm, idx_v, ones_v, acc_sh,
                sem):
    c = lax.axis_index("c")
    s = lax.axis_index("s")
    wid = c * NS + s
    pltpu.sync_copy(zeros_hbm, acc_sh.at[pl.ds(s * RPT, RPT)])
    pltpu.sync_copy(row_hbm.at[wid], idx_v)
    pltpu.sync_copy(ones_hbm, ones_v)
    plsc.subcore_barrier()

    def body(i, _):
        descs = [
            pltpu.async_copy(ones_v, acc_sh.at[idx_v.at[i * KD + k]],
                             sem, add=True)
            for k in range(KD)
        ]
        for d in descs:
            d.wait()
        return _

    lax.fori_loop(0, C // KD, body, None)
    plsc.subcore_barrier()
    pltpu.sync_copy(acc_sh.at[pl.ds(s * RPT, RPT)], out_hbm.at[c, s])


def _make_conv_kernel(Dr):
    @functools.partial(
        pl.kernel,
        out_type=jax.ShapeDtypeStruct((NC, NS, RPT, Dr), jnp.float32),
        mesh=_mesh,
        compiler_params=_sc_params,
        scratch_types=[
            pltpu.VMEM((C, B), jnp.int32),
            pltpu.VMEM((C, B), jnp.int32),
            pltpu.VMEM((3, KC, B, Dr), jnp.float32),
            pltpu.VMEM_SHARED((N, Dr), jnp.float32),
            pltpu.SemaphoreType.DMA,
            pltpu.SemaphoreType.DMA,
            pltpu.SemaphoreType.DMA,
            pltpu.SemaphoreType.DMA,
            pltpu.SemaphoreType.DMA,
            pltpu.SemaphoreType.DMA,
        ],
    )
    def _conv_kernel(table_hbm, row_hbm, col_hbm, zeros_hbm, out_hbm,
                     row_v, col_v, buf_v, acc_sh,
                     sg0, sg1, sg2, ss0, ss1, ss2):
        c = lax.axis_index("c")
        s = lax.axis_index("s")
        wid = c * NS + s
        sg = (sg0, sg1, sg2)
        ss = (ss0, ss1, ss2)

        pltpu.sync_copy(zeros_hbm, acc_sh.at[pl.ds(s * RPT, RPT)])
        pltpu.sync_copy(row_hbm.at[wid], row_v)
        pltpu.sync_copy(col_hbm.at[wid], col_v)
        plsc.subcore_barrier()

        T = C // KC

        def issue_g(t, m):
            for k in range(KC):
                pltpu.async_copy(table_hbm.at[row_v.at[t * KC + k]],
                                 buf_v.at[m, k], sg[m])

        def drain_g(t, m):
            for k in range(KC):
                pltpu.make_async_copy(table_hbm.at[row_v.at[t * KC + k]],
                                      buf_v.at[m, k], sg[m]).wait()

        def issue_s(t, m):
            for k in range(KC):
                pltpu.async_copy(buf_v.at[m, k],
                                 acc_sh.at[col_v.at[t * KC + k]],
                                 ss[m], add=True)

        def drain_s(t, m):
            for k in range(KC):
                pltpu.make_async_copy(buf_v.at[m, k],
                                      acc_sh.at[col_v.at[t * KC + k]],
                                      ss[m]).wait()

        issue_g(0, 0)
        issue_g(1, 1)
        drain_g(0, 0)
        issue_s(0, 0)
        issue_g(2, 2)

        def body(i, _):
            t = 3 * i + 1
            for dm in range(3):
                m = (1 + dm) % 3
                drain_g(t + dm, m)
                issue_s(t + dm, m)
                drain_s(t + dm - 1, (m + 2) % 3)
                issue_g(t + dm + 2, (m + 2) % 3)
            return _

        lax.fori_loop(0, (T - 4) // 3, body, None)
        for t in (T - 3, T - 2, T - 1):
            m = t % 3
            drain_g(t, m)
            issue_s(t, m)
            drain_s(t - 1, (m + 2) % 3)
            if t + 2 < T:
                issue_g(t + 2, (m + 2) % 3)
        drain_s(T - 1, (T - 1) % 3)

        plsc.subcore_barrier()
        pltpu.sync_copy(acc_sh.at[pl.ds(s * RPT, RPT)], out_hbm.at[c, s])

    return _conv_kernel


_conv32 = _make_conv_kernel(32)
_conv8 = _make_conv_kernel(8)


def _iota2(shape, dim):
    return lax.broadcasted_iota(jnp.int32, shape, dim)


def _dis32(d0_ref, d1_ref):
    i = _iota2((128, 128), 0)
    j = _iota2((128, 128), 1)
    r32 = ((i % 32 == 0) & (j // 32 == i // 32)).astype(jnp.float32)
    dsum = jnp.dot(d0_ref[...] + d1_ref[...], r32,
                   preferred_element_type=jnp.float32)
    return lax.rsqrt(dsum + 1.0)


def _tc1_body(xp_ref, bd1_ref, b1p_ref, d0_ref, d1_ref, ht_ref):
    h = jnp.dot(xp_ref[...], bd1_ref[...],
                preferred_element_type=jnp.float32) + b1p_ref[...]
    ht_ref[...] = h * _dis32(d0_ref, d1_ref)


def _tc2_body(p0_ref, p1_ref, ht1_ref, d0_ref, d1_ref, bd2_ref, b2q_ref,
              ht2_ref, dis8_ref):
    dis32 = _dis32(d0_ref, d1_ref)
    s = p0_ref[...] + p1_ref[...] + ht1_ref[...]
    out1 = jnp.maximum(dis32 * s, 0.0)
    h2 = jnp.dot(out1, bd2_ref[...],
                 preferred_element_type=jnp.float32) + b2q_ref[...]
    i = _iota2((128, 32), 0)
    j = _iota2((128, 32), 1)
    s8 = (i == 32 * (j // 8)).astype(jnp.float32)
    dis8 = jnp.dot(dis32, s8, preferred_element_type=jnp.float32)
    ht2_ref[...] = h2 * dis8
    dis8_ref[...] = dis8


def _tc3_body(q0_ref, q1_ref, ht2_ref, dis8_ref, x1p_ref, wl_ref, bl_ref,
              z_ref, emb_ref):
    s = q0_ref[...] + q1_ref[...] + ht2_ref[...]
    out2 = dis8_ref[...] * s
    m = jnp.max(out2, axis=1, keepdims=True)
    e = jnp.exp(out2 - m)
    i = _iota2((128, 128), 0)
    j = _iota2((128, 128), 1)
    g8 = ((i // 8) == (j // 8)).astype(jnp.float32)
    ssum = jnp.dot(e, g8, preferred_element_type=jnp.float32)
    emb = (out2 - m) - jnp.log(ssum)
    wl = wl_ref[...]
    it = _iota2((8, 128), 0)
    jt = _iota2((8, 128), 1)
    tile8 = (jt % 8 == it).astype(jnp.float32)
    wlp = jnp.dot(wl[:, 0:8], tile8, preferred_element_type=jnp.float32)
    ig = _iota2((128, 16), 0)
    jg = _iota2((128, 16), 1)
    gsel = ((ig // 8) == jg).astype(jnp.float32)
    zq = jnp.dot(emb * wlp, gsel, preferred_element_type=jnp.float32)
    z = zq + x1p_ref[...] * wl[:, 8:9] + bl_ref[...]
    z_ref[...] = jnp.maximum(z, 0.0)
    emb_ref[...] = emb


def kernel(x, edge_index, x1, W1, b1, W2, b2, Wl, bl):
    f32 = jnp.float32
    row3 = edge_index[0].reshape(NW, C, B)
    col3 = edge_index[1].reshape(NW, C, B)
    zeros32 = jnp.zeros((RPT, 32), f32)
    zeros8 = jnp.zeros((RPT, 8), f32)
    ones_hbm = jnp.zeros((B, 32), f32).at[:, 0].set(1.0)

    degp = _deg_kernel(row3, zeros32, ones_hbm)
    degp3 = degp.reshape(NC, N // 4, 128)
    d0p, d1p = degp3[0], degp3[1]

    bd1 = jax.scipy.linalg.block_diag(*([W1.T] * 4))
    bd2 = jax.scipy.linalg.block_diag(*([W2.T] * 4))
    b1p = jnp.tile(b1, 4).reshape(1, 128)
    b2q = jnp.tile(b2, 4).reshape(1, 32)

    ht1p = pl.pallas_call(
        _tc1_body,
        out_shape=jax.ShapeDtypeStruct((N // 4, 128), f32),
    )(x.reshape(N // 4, 512), bd1, b1p, d0p, d1p)

    p3 = _conv32(ht1p.reshape(N, 32), row3, col3,
                 zeros32).reshape(NC, N // 4, 128)

    ht2q, dis8q = pl.pallas_call(
        _tc2_body,
        out_shape=[jax.ShapeDtypeStruct((N // 4, 32), f32),
                   jax.ShapeDtypeStruct((N // 4, 32), f32)],
    )(p3[0], p3[1], ht1p, d0p, d1p, bd2, b2q)

    ht2lin = ht2q.reshape(N, 8)
    q3 = _conv8(ht2lin, row3, col3, zeros8).reshape(NC, N // 16, 128)

    z16, embp = pl.pallas_call(
        _tc3_body,
        out_shape=[jax.ShapeDtypeStruct((N // 16, 16), f32),
                   jax.ShapeDtypeStruct((N // 16, 128), f32)],
    )(q3[0], q3[1],
      ht2q.reshape(N // 16, 128), dis8q.reshape(N // 16, 128),
      x1.reshape(N // 16, 16), Wl, bl.reshape(1, 1))

    return (z16.reshape(N, 1), embp.reshape(N, 8))
